# Initial kernel scaffold; baseline (speedup 1.0000x reference)
import jax
import jax.numpy as jnp
from jax.experimental import pallas as pl


def kernel(input_ids, table):
    raise NotImplementedError("write your pallas kernel here")


# speedup vs baseline: 1.4780x; 1.4780x over previous
import functools

import jax
import jax.numpy as jnp
from jax import lax
from jax.experimental import pallas as pl
from jax.experimental.pallas import tpu as pltpu
from jax.experimental.pallas import tpu_sc as plsc

_NUM_CORES = 2
_NUM_SUBCORES = 16
_NUM_WORKERS = _NUM_CORES * _NUM_SUBCORES


def _gather_call(B, D, b_per_w, chunk):
    n_chunks = b_per_w // chunk
    mesh = plsc.VectorSubcoreMesh(core_axis_name="c", subcore_axis_name="s")

    @functools.partial(
        pl.kernel,
        mesh=mesh,
        out_type=jax.ShapeDtypeStruct((B, D), jnp.float32),
        scratch_types=[
            pltpu.VMEM((chunk,), jnp.int32),
            pltpu.VMEM((chunk, D), jnp.float32),
            pltpu.SemaphoreType.DMA,
        ],
        compiler_params=pltpu.CompilerParams(use_tc_tiling_on_sc=False),
    )
    def gather_kernel(table_hbm, idx_hbm, out_hbm, idx_v, rows_v, sem):
        wid = lax.axis_index("s") * _NUM_CORES + lax.axis_index("c")
        base = wid * b_per_w

        def body(i, _):
            off = base + i * chunk
            pltpu.sync_copy(idx_hbm.at[pl.ds(off, chunk)], idx_v)
            pltpu.async_copy(table_hbm.at[idx_v], rows_v, sem).wait()
            pltpu.sync_copy(rows_v, out_hbm.at[pl.ds(off, chunk)])
            return ()

        lax.fori_loop(0, n_chunks, body, ())

    return gather_kernel


def kernel(input_ids, table):
    batch, seq = input_ids.shape
    _, D = table.shape
    idx = input_ids.reshape(-1).astype(jnp.int32)
    B = batch * seq
    b_per_w = B // _NUM_WORKERS
    out = _gather_call(B, D, b_per_w, chunk=1600)(table, idx)
    return out.reshape(batch, seq, D)


# speedup vs baseline: 1.4926x; 1.0098x over previous
import functools

import jax
import jax.numpy as jnp
from jax import lax
from jax.experimental import pallas as pl
from jax.experimental.pallas import tpu as pltpu
from jax.experimental.pallas import tpu_sc as plsc

_NUM_CORES = 2
_NUM_SUBCORES = 16
_NUM_WORKERS = _NUM_CORES * _NUM_SUBCORES
_NBUF = 2


def _gather_call(B, D, b_per_w, chunk):
    n_chunks = b_per_w // chunk
    n_groups = n_chunks // _NBUF
    mesh = plsc.VectorSubcoreMesh(core_axis_name="c", subcore_axis_name="s")

    @functools.partial(
        pl.kernel,
        mesh=mesh,
        out_type=jax.ShapeDtypeStruct((B, D), jnp.float32),
        scratch_types=(
            [pltpu.VMEM((chunk,), jnp.int32) for _ in range(_NBUF)]
            + [pltpu.VMEM((chunk, D), jnp.float32) for _ in range(_NBUF)]
            + [pltpu.SemaphoreType.DMA for _ in range(3 * _NBUF)]
        ),
        compiler_params=pltpu.CompilerParams(use_tc_tiling_on_sc=False),
    )
    def gather_kernel(table_hbm, idx_hbm, out_hbm, *scratch):
        idx_v = scratch[:_NBUF]
        rows_v = scratch[_NBUF:2 * _NBUF]
        s_i = scratch[2 * _NBUF:3 * _NBUF]
        s_g = scratch[3 * _NBUF:4 * _NBUF]
        s_o = scratch[4 * _NBUF:5 * _NBUF]

        wid = lax.axis_index("s") * _NUM_CORES + lax.axis_index("c")
        base = wid * b_per_w

        def start_idx(b, g):
            pltpu.async_copy(idx_hbm.at[pl.ds(base + g * chunk, chunk)],
                             idx_v[b], s_i[b])

        def wait_idx(b):
            pltpu.make_async_copy(idx_hbm.at[pl.ds(0, chunk)],
                                  idx_v[b], s_i[b]).wait()

        def start_gather(b):
            pltpu.async_copy(table_hbm.at[idx_v[b]], rows_v[b], s_g[b])

        def wait_gather(b):
            pltpu.make_async_copy(table_hbm.at[idx_v[b]],
                                  rows_v[b], s_g[b]).wait()

        def start_out(b, g):
            pltpu.async_copy(rows_v[b],
                             out_hbm.at[pl.ds(base + g * chunk, chunk)],
                             s_o[b])

        def wait_out(b):
            pltpu.make_async_copy(rows_v[b],
                                  out_hbm.at[pl.ds(0, chunk)], s_o[b]).wait()

        for b in range(_NBUF):
            start_idx(b, b)
        for b in range(_NBUF):
            wait_idx(b)
            start_gather(b)
            wait_gather(b)
            start_out(b, b)
            start_idx(b, b + _NBUF)

        def group(i, _):
            o = i * _NBUF
            for b in range(_NBUF):
                g = o + b
                wait_out(b)
                wait_idx(b)
                start_gather(b)
                wait_gather(b)
                start_out(b, g)

                @pl.when(g + _NBUF < n_chunks)
                def _():
                    start_idx(b, g + _NBUF)
            return ()

        lax.fori_loop(1, n_groups, group, ())

        for b in range(_NBUF):
            wait_out(b)

    return gather_kernel


def kernel(input_ids, table):
    batch, seq = input_ids.shape
    _, D = table.shape
    idx = input_ids.reshape(-1).astype(jnp.int32)
    B = batch * seq
    b_per_w = B // _NUM_WORKERS
    out = _gather_call(B, D, b_per_w, chunk=1600)(table, idx)
    return out.reshape(batch, seq, D)
